# baseline (speedup 1.0000x reference)
import functools
import math

import jax
import jax.numpy as jnp
from jax import lax
from jax.experimental import pallas as pl
from jax.experimental.pallas import tpu as pltpu
from jax.experimental.pallas import tpu_sc as plsc

D_MODEL = 64
SCALE = math.sqrt(D_MODEL)
LANES = 16
S_LEN = 200
B4 = 4096
VOCAB = 1000000
VOCAB2 = VOCAB // 2

_info = plsc.get_sparse_core_info()
NUM_CORES = _info.num_cores
NUM_SUBCORES = _info.num_subcores
NW = NUM_CORES * NUM_SUBCORES

_mesh = plsc.VectorSubcoreMesh(core_axis_name="c", subcore_axis_name="s")

_params = pltpu.CompilerParams(use_tc_tiling_on_sc=True, needs_layout_passes=False)


PANC = 256
NPAN = VOCAB // PANC
PROWS = PANC // 2
TAIL_ROWS = (VOCAB - NPAN * PANC) // 2
NB_A = 3
NSLOT_A = (-(-(-(-NPAN // NW)) // NB_A)) * NB_A


@functools.partial(
    pl.kernel,
    mesh=_mesh,
    out_type=jax.ShapeDtypeStruct((VOCAB2, 128), jnp.float32),
    scratch_types=[
        [pltpu.VMEM((D_MODEL, PANC), jnp.float32) for _ in range(NB_A)],
        [pltpu.VMEM((PROWS, 128), jnp.float32) for _ in range(NB_A)],
        [pltpu.SemaphoreType.DMA for _ in range(NB_A)],
        [pltpu.SemaphoreType.DMA for _ in range(NB_A)],
    ],
    compiler_params=_params,
)
def _build_table(tabt_hbm, tail_hbm, tab2_hbm, pan, pant, gsem, osem):
    wid = lax.axis_index("s") * NUM_CORES + lax.axis_index("c")
    iot = lax.iota(jnp.int32, 16)

    def panel_of(t):
        return wid + t * NW

    def fire_read(b, t):
        p = panel_of(t)

        @pl.when(p < NPAN)
        def _():
            pltpu.async_copy(
                tabt_hbm.at[:, pl.ds(p * PANC, PANC)], pan[b], gsem[b]
            )

    def wait_read(b, t):
        p = panel_of(t)

        @pl.when(p < NPAN)
        def _():
            pltpu.make_async_copy(
                tabt_hbm.at[:, pl.ds(p * PANC, PANC)], pan[b], gsem[b]
            ).wait()

    def fire_write(b, t):
        p = panel_of(t)

        @pl.when(p < NPAN)
        def _():
            pltpu.async_copy(
                pant[b], tab2_hbm.at[pl.ds(p * PROWS, PROWS)], osem[b]
            )

    def wait_write(b, t):
        p = panel_of(t)

        @pl.when(p < NPAN)
        def _():
            pltpu.make_async_copy(
                pant[b], tab2_hbm.at[pl.ds(p * PROWS, PROWS)], osem[b]
            ).wait()

    def transform(b, t):
        p = panel_of(t)

        @pl.when(p < NPAN)
        def _():
            for c0 in range(0, PANC, LANES):
                cv = iot + c0
                qv = cv >> 1
                pv = (cv & 1) << 6

                @plsc.parallel_loop(0, D_MODEL, unroll=8)
                def dloop(d0):
                    dv = (iot + d0) & (D_MODEL - 1)
                    g = plsc.load_gather(pan[b], [dv, cv])
                    plsc.store_scatter(pant[b], [qv, pv + dv], g * SCALE)

    for b in range(NB_A):
        fire_read(b, b)

    def outer(t0i, carry):
        t0 = t0i * NB_A
        for b in range(NB_A):
            t = t0 + b
            wait_read(b, t)

            @pl.when(t >= NB_A)
            def _():
                wait_write(b, t - NB_A)

            transform(b, t)
            fire_write(b, t)
            fire_read(b, t + NB_A)
        return carry

    lax.fori_loop(0, NSLOT_A // NB_A, outer, 0)

    for b in range(NB_A):
        wait_write(b, NSLOT_A - NB_A + b)

    @pl.when(wid == 0)
    def _():
        pltpu.sync_copy(tail_hbm, pant[0].at[pl.ds(0, TAIL_ROWS)])
        pltpu.sync_copy(
            pant[0].at[pl.ds(0, TAIL_ROWS)],
            tab2_hbm.at[pl.ds(NPAN * PROWS, TAIL_ROWS)],
        )


BLK = B4 // NW
NBUF = 4
NTASK = S_LEN


@functools.partial(
    pl.kernel,
    mesh=_mesh,
    out_type=jax.ShapeDtypeStruct((S_LEN, D_MODEL, B4), jnp.float32),
    scratch_types=[
        [pltpu.VMEM((BLK,), jnp.int32) for _ in range(NBUF)],
        [pltpu.VMEM((BLK,), jnp.int32) for _ in range(NBUF)],
        [pltpu.VMEM((BLK, 128), jnp.float32) for _ in range(NBUF)],
        [pltpu.VMEM((D_MODEL, BLK), jnp.float32) for _ in range(NBUF)],
        [pltpu.SemaphoreType.DMA for _ in range(NBUF)],
        [pltpu.SemaphoreType.DMA for _ in range(NBUF)],
    ],
    compiler_params=_params,
)
def _emb_lookup(xt_hbm, tab2_hbm, out_hbm, idxr, idx2, rows, tblk, gsem, osem):
    wid = lax.axis_index("s") * NUM_CORES + lax.axis_index("c")
    col0 = wid * BLK
    iot = lax.iota(jnp.int32, 16)

    def stage_and_gather(b, t):
        pltpu.sync_copy(xt_hbm.at[t, pl.ds(col0, BLK)], idxr[b])
        for j0 in range(BLK // LANES):
            sl = pl.ds(j0 * LANES, LANES)
            idx2[b][sl] = idxr[b][sl] >> 1
        pltpu.async_copy(tab2_hbm.at[idx2[b]], rows[b], gsem[b])

    def wait_gather(b):
        pltpu.make_async_copy(tab2_hbm.at[idx2[b]], rows[b], gsem[b]).wait()

    def fire_write(b, t):
        pltpu.async_copy(tblk[b], out_hbm.at[t, :, pl.ds(col0, BLK)], osem[b])

    def wait_write(b, t):
        pltpu.make_async_copy(
            tblk[b], out_hbm.at[t, :, pl.ds(col0, BLK)], osem[b]
        ).wait()

    def transform(b):
        for j0 in range(BLK // LANES):
            sl = pl.ds(j0 * LANES, LANES)
            idxv = idxr[b][sl]
            pv = (idxv & 1) << 6
            jv = iot + j0 * LANES

            @plsc.parallel_loop(0, D_MODEL, unroll=8)
            def dloop(d0):
                dv = (iot + d0) & (D_MODEL - 1)
                g = plsc.load_gather(rows[b], [jv, pv + dv])
                plsc.store_scatter(tblk[b], [dv, jv], g)

    for b in range(NBUF):
        stage_and_gather(b, b)

    def outer(t0i, carry):
        t0 = t0i * NBUF
        for b in range(NBUF):
            t = t0 + b
            wait_gather(b)

            @pl.when(t >= NBUF)
            def _():
                wait_write(b, t - NBUF)

            transform(b)
            fire_write(b, t)

            @pl.when(t + NBUF < NTASK)
            def _():
                stage_and_gather(b, t + NBUF)

        return carry

    lax.fori_loop(0, NTASK // NBUF, outer, 0)

    for b in range(NBUF):
        wait_write(b, NTASK - NBUF + b)


def kernel(x, table):
    xt = x.astype(jnp.int32).T
    tabt = table.T
    tail = (table[NPAN * PANC :, :] * SCALE).reshape(TAIL_ROWS, 128)
    tab2 = _build_table(tabt, tail)
    out_phys = _emb_lookup(xt, tab2)
    return jnp.transpose(out_phys, (2, 0, 1))


# speedup vs baseline: 1.1919x; 1.1919x over previous
import functools
import math

import jax
import jax.numpy as jnp
from jax import lax
from jax.experimental import pallas as pl
from jax.experimental.pallas import tpu as pltpu
from jax.experimental.pallas import tpu_sc as plsc

D_MODEL = 64
SCALE = math.sqrt(D_MODEL)
LANES = 16
S_LEN = 200
B4 = 4096
VOCAB = 1000000
VOCAB2 = VOCAB // 2

_info = plsc.get_sparse_core_info()
NUM_CORES = _info.num_cores
NUM_SUBCORES = _info.num_subcores
NW = NUM_CORES * NUM_SUBCORES

_mesh = plsc.VectorSubcoreMesh(core_axis_name="c", subcore_axis_name="s")

_params = pltpu.CompilerParams(use_tc_tiling_on_sc=True, needs_layout_passes=False)


PANC = 128
NPAN = VOCAB // PANC
PROWS = PANC // 2
TAIL_ROWS = (VOCAB - NPAN * PANC) // 2
NB_A = 4
NSLOT_A = (-(-(-(-NPAN // NW)) // NB_A)) * NB_A


@functools.partial(
    pl.kernel,
    mesh=_mesh,
    out_type=jax.ShapeDtypeStruct((VOCAB2, 128), jnp.float32),
    scratch_types=[
        [pltpu.VMEM((D_MODEL, PANC), jnp.float32) for _ in range(NB_A)],
        [pltpu.VMEM((PROWS, 128), jnp.float32) for _ in range(NB_A)],
        [pltpu.SemaphoreType.DMA for _ in range(NB_A)],
        [pltpu.SemaphoreType.DMA for _ in range(NB_A)],
    ],
    compiler_params=_params,
)
def _build_table(tabt_hbm, tail_hbm, tab2_hbm, pan, pant, gsem, osem):
    wid = lax.axis_index("s") * NUM_CORES + lax.axis_index("c")
    iot = lax.iota(jnp.int32, 16)

    def panel_of(t):
        return wid + t * NW

    def fire_read(b, t):
        p = panel_of(t)

        @pl.when(p < NPAN)
        def _():
            pltpu.async_copy(
                tabt_hbm.at[:, pl.ds(p * PANC, PANC)], pan[b], gsem[b]
            )

    def wait_read(b, t):
        p = panel_of(t)

        @pl.when(p < NPAN)
        def _():
            pltpu.make_async_copy(
                tabt_hbm.at[:, pl.ds(p * PANC, PANC)], pan[b], gsem[b]
            ).wait()

    def fire_write(b, t):
        p = panel_of(t)

        @pl.when(p < NPAN)
        def _():
            pltpu.async_copy(
                pant[b], tab2_hbm.at[pl.ds(p * PROWS, PROWS)], osem[b]
            )

    def wait_write(b, t):
        p = panel_of(t)

        @pl.when(p < NPAN)
        def _():
            pltpu.make_async_copy(
                pant[b], tab2_hbm.at[pl.ds(p * PROWS, PROWS)], osem[b]
            ).wait()

    def transform(b, t):
        p = panel_of(t)

        @pl.when(p < NPAN)
        def _():
            for c0 in range(0, PANC, LANES):
                cv = iot + c0
                qv = cv >> 1
                pv = (cv & 1) << 6

                @plsc.parallel_loop(0, D_MODEL, unroll=8)
                def dloop(d0):
                    dv = (iot + d0) & (D_MODEL - 1)
                    g = plsc.load_gather(pan[b], [dv, cv])
                    plsc.store_scatter(pant[b], [qv, pv + dv], g * SCALE)

    for b in range(NB_A):
        fire_read(b, b)

    def outer(t0i, carry):
        t0 = t0i * NB_A
        for b in range(NB_A):
            t = t0 + b
            wait_read(b, t)

            @pl.when(t >= NB_A)
            def _():
                wait_write(b, t - NB_A)

            transform(b, t)
            fire_write(b, t)
            fire_read(b, t + NB_A)
        return carry

    lax.fori_loop(0, NSLOT_A // NB_A, outer, 0)

    for b in range(NB_A):
        wait_write(b, NSLOT_A - NB_A + b)

    @pl.when(wid == 0)
    def _():
        pltpu.sync_copy(tail_hbm, pant[0].at[pl.ds(0, TAIL_ROWS)])
        pltpu.sync_copy(
            pant[0].at[pl.ds(0, TAIL_ROWS)],
            tab2_hbm.at[pl.ds(NPAN * PROWS, TAIL_ROWS)],
        )


BLK = B4 // NW
NBUF = 4
NTASK = S_LEN


@functools.partial(
    pl.kernel,
    mesh=_mesh,
    out_type=jax.ShapeDtypeStruct((S_LEN, D_MODEL, B4), jnp.float32),
    scratch_types=[
        pltpu.VMEM((NTASK, BLK), jnp.int32),
        [pltpu.VMEM((BLK,), jnp.int32) for _ in range(NBUF)],
        [pltpu.VMEM((BLK, 128), jnp.float32) for _ in range(NBUF)],
        [pltpu.VMEM((D_MODEL, BLK), jnp.float32) for _ in range(NBUF)],
        [pltpu.SemaphoreType.DMA for _ in range(NBUF)],
        [pltpu.SemaphoreType.DMA for _ in range(NBUF)],
    ],
    compiler_params=_params,
)
def _emb_lookup(xt_hbm, tab2_hbm, out_hbm, idxr, idx2, rows, tblk, gsem, osem):
    wid = lax.axis_index("s") * NUM_CORES + lax.axis_index("c")
    col0 = wid * BLK
    iot = lax.iota(jnp.int32, 16)

    pltpu.sync_copy(xt_hbm.at[:, pl.ds(col0, BLK)], idxr)

    def stage_and_gather(b, t):
        for j0 in range(BLK // LANES):
            sl = pl.ds(j0 * LANES, LANES)
            idx2[b][sl] = idxr[t, sl] >> 1
        pltpu.async_copy(tab2_hbm.at[idx2[b]], rows[b], gsem[b])

    def wait_gather(b):
        pltpu.make_async_copy(tab2_hbm.at[idx2[b]], rows[b], gsem[b]).wait()

    def fire_write(b, t):
        pltpu.async_copy(tblk[b], out_hbm.at[t, :, pl.ds(col0, BLK)], osem[b])

    def wait_write(b, t):
        pltpu.make_async_copy(
            tblk[b], out_hbm.at[t, :, pl.ds(col0, BLK)], osem[b]
        ).wait()

    def transform(b, t):
        for j0 in range(BLK // LANES):
            sl = pl.ds(j0 * LANES, LANES)
            idxv = idxr[t, sl]
            pv = (idxv & 1) << 6
            jv = iot + j0 * LANES

            @plsc.parallel_loop(0, D_MODEL, unroll=8)
            def dloop(d0):
                dv = (iot + d0) & (D_MODEL - 1)
                g = plsc.load_gather(rows[b], [jv, pv + dv])
                plsc.store_scatter(tblk[b], [dv, jv], g)

    for b in range(NBUF):
        stage_and_gather(b, b)

    def outer(t0i, carry):
        t0 = t0i * NBUF
        for b in range(NBUF):
            t = t0 + b
            wait_gather(b)

            @pl.when(t >= NBUF)
            def _():
                wait_write(b, t - NBUF)

            transform(b, t)
            fire_write(b, t)

            @pl.when(t + NBUF < NTASK)
            def _():
                stage_and_gather(b, t + NBUF)

        return carry

    lax.fori_loop(0, NTASK // NBUF, outer, 0)

    for b in range(NBUF):
        wait_write(b, NTASK - NBUF + b)


def kernel(x, table):
    xt = x.astype(jnp.int32).T
    tabt = table.T
    tail = (table[NPAN * PANC :, :] * SCALE).reshape(TAIL_ROWS, 128)
    tab2 = _build_table(tabt, tail)
    out_phys = _emb_lookup(xt, tab2)
    return jnp.transpose(out_phys, (2, 0, 1))


# speedup vs baseline: 1.3148x; 1.1031x over previous
import functools
import math

import jax
import jax.numpy as jnp
from jax import lax
from jax.experimental import pallas as pl
from jax.experimental.pallas import tpu as pltpu
from jax.experimental.pallas import tpu_sc as plsc

D_MODEL = 64
SCALE = math.sqrt(D_MODEL)
LANES = 16
S_LEN = 200
B4 = 4096
VOCAB = 1000000
VOCAB2 = VOCAB // 2

_info = plsc.get_sparse_core_info()
NUM_CORES = _info.num_cores
NUM_SUBCORES = _info.num_subcores
NW = NUM_CORES * NUM_SUBCORES

_mesh = plsc.VectorSubcoreMesh(core_axis_name="c", subcore_axis_name="s")

_params = pltpu.CompilerParams(use_tc_tiling_on_sc=True, needs_layout_passes=False)


PANC = 128
NPAN = VOCAB // PANC
PROWS = PANC // 2
TAIL_ROWS = (VOCAB - NPAN * PANC) // 2
NB_A = 4
NSLOT_A = (-(-(-(-NPAN // NW)) // NB_A)) * NB_A


@functools.partial(
    pl.kernel,
    mesh=_mesh,
    out_type=jax.ShapeDtypeStruct((VOCAB2, 128), jnp.float32),
    scratch_types=[
        [pltpu.VMEM((D_MODEL, PANC), jnp.float32) for _ in range(NB_A)],
        [pltpu.VMEM((PROWS, 128), jnp.float32) for _ in range(NB_A)],
        [pltpu.SemaphoreType.DMA for _ in range(NB_A)],
        [pltpu.SemaphoreType.DMA for _ in range(NB_A)],
    ],
    compiler_params=_params,
)
def _build_table(tabt_hbm, tail_hbm, tab2_hbm, pan, pant, gsem, osem):
    wid = lax.axis_index("s") * NUM_CORES + lax.axis_index("c")
    iot = lax.iota(jnp.int32, 16)

    def panel_of(t):
        return wid + t * NW

    def fire_read(b, t):
        p = panel_of(t)

        @pl.when(p < NPAN)
        def _():
            pltpu.async_copy(
                tabt_hbm.at[:, pl.ds(p * PANC, PANC)], pan[b], gsem[b]
            )

    def wait_read(b, t):
        p = panel_of(t)

        @pl.when(p < NPAN)
        def _():
            pltpu.make_async_copy(
                tabt_hbm.at[:, pl.ds(p * PANC, PANC)], pan[b], gsem[b]
            ).wait()

    def fire_write(b, t):
        p = panel_of(t)

        @pl.when(p < NPAN)
        def _():
            pltpu.async_copy(
                pant[b], tab2_hbm.at[pl.ds(p * PROWS, PROWS)], osem[b]
            )

    def wait_write(b, t):
        p = panel_of(t)

        @pl.when(p < NPAN)
        def _():
            pltpu.make_async_copy(
                pant[b], tab2_hbm.at[pl.ds(p * PROWS, PROWS)], osem[b]
            ).wait()

    def transform(b, t):
        p = panel_of(t)

        @pl.when(p < NPAN)
        def _():
            for c0 in range(0, PANC, LANES):
                cv = iot + c0
                qv = cv >> 1
                pv = (cv & 1) << 6

                @plsc.parallel_loop(0, D_MODEL, unroll=4)
                def dloop(d0):
                    dv = (iot + d0) & (D_MODEL - 1)
                    g = plsc.load_gather(pan[b], [dv, cv])
                    plsc.store_scatter(pant[b], [qv, pv + dv], g * SCALE)

    for b in range(NB_A):
        fire_read(b, b)

    def outer(t0i, carry):
        t0 = t0i * NB_A
        for b in range(NB_A):
            t = t0 + b
            wait_read(b, t)

            @pl.when(t >= NB_A)
            def _():
                wait_write(b, t - NB_A)

            transform(b, t)
            fire_write(b, t)
            fire_read(b, t + NB_A)
        return carry

    lax.fori_loop(0, NSLOT_A // NB_A, outer, 0)

    for b in range(NB_A):
        wait_write(b, NSLOT_A - NB_A + b)

    @pl.when(wid == 0)
    def _():
        pltpu.sync_copy(tail_hbm, pant[0].at[pl.ds(0, TAIL_ROWS)])
        pltpu.sync_copy(
            pant[0].at[pl.ds(0, TAIL_ROWS)],
            tab2_hbm.at[pl.ds(NPAN * PROWS, TAIL_ROWS)],
        )


BLK = B4 // NW
NBUF = 4
NTASK = S_LEN


@functools.partial(
    pl.kernel,
    mesh=_mesh,
    out_type=jax.ShapeDtypeStruct((S_LEN, D_MODEL, B4), jnp.float32),
    scratch_types=[
        pltpu.VMEM((NTASK, BLK), jnp.int32),
        [pltpu.VMEM((BLK,), jnp.int32) for _ in range(NBUF)],
        [pltpu.VMEM((BLK, 128), jnp.float32) for _ in range(NBUF)],
        [pltpu.VMEM((D_MODEL, BLK), jnp.float32) for _ in range(NBUF)],
        [pltpu.SemaphoreType.DMA for _ in range(NBUF)],
        [pltpu.SemaphoreType.DMA for _ in range(NBUF)],
    ],
    compiler_params=_params,
)
def _emb_lookup(xt_hbm, tab2_hbm, out_hbm, idxr, idx2, rows, tblk, gsem, osem):
    wid = lax.axis_index("s") * NUM_CORES + lax.axis_index("c")
    col0 = wid * BLK
    iot = lax.iota(jnp.int32, 16)

    pltpu.sync_copy(xt_hbm.at[:, pl.ds(col0, BLK)], idxr)

    def stage_and_gather(b, t):
        for j0 in range(BLK // LANES):
            sl = pl.ds(j0 * LANES, LANES)
            idx2[b][sl] = idxr[t, sl] >> 1
        pltpu.async_copy(tab2_hbm.at[idx2[b]], rows[b], gsem[b])

    def wait_gather(b):
        pltpu.make_async_copy(tab2_hbm.at[idx2[b]], rows[b], gsem[b]).wait()

    def fire_write(b, t):
        pltpu.async_copy(tblk[b], out_hbm.at[t, :, pl.ds(col0, BLK)], osem[b])

    def wait_write(b, t):
        pltpu.make_async_copy(
            tblk[b], out_hbm.at[t, :, pl.ds(col0, BLK)], osem[b]
        ).wait()

    def transform(b, t):
        for j0 in range(BLK // LANES):
            sl = pl.ds(j0 * LANES, LANES)
            idxv = idxr[t, sl]
            pv = (idxv & 1) << 6
            jv = iot + j0 * LANES

            @plsc.parallel_loop(0, D_MODEL, unroll=4)
            def dloop(d0):
                dv = (iot + d0) & (D_MODEL - 1)
                g = plsc.load_gather(rows[b], [jv, pv + dv])
                plsc.store_scatter(tblk[b], [dv, jv], g)

    for b in range(NBUF):
        stage_and_gather(b, b)

    def outer(t0i, carry):
        t0 = t0i * NBUF
        for b in range(NBUF):
            t = t0 + b
            wait_gather(b)

            @pl.when(t >= NBUF)
            def _():
                wait_write(b, t - NBUF)

            transform(b, t)
            fire_write(b, t)

            @pl.when(t + NBUF < NTASK)
            def _():
                stage_and_gather(b, t + NBUF)

        return carry

    lax.fori_loop(0, NTASK // NBUF, outer, 0)

    for b in range(NBUF):
        wait_write(b, NTASK - NBUF + b)


def kernel(x, table):
    xt = x.astype(jnp.int32).T
    tabt = table.T
    tail = (table[NPAN * PANC :, :] * SCALE).reshape(TAIL_ROWS, 128)
    tab2 = _build_table(tabt, tail)
    out_phys = _emb_lookup(xt, tab2)
    return jnp.transpose(out_phys, (2, 0, 1))
